# Initial kernel scaffold; baseline (speedup 1.0000x reference)
import jax
import jax.numpy as jnp
from jax.experimental import pallas as pl


def kernel(x_node, x_edge, edge_index, W, b):
    raise NotImplementedError("write your pallas kernel here")


# speedup vs baseline: 7.2732x; 7.2732x over previous
import functools

import jax
import jax.numpy as jnp
from jax import lax
from jax.experimental import pallas as pl
from jax.experimental.pallas import tpu as pltpu
from jax.experimental.pallas import tpu_sc as plsc

N_NODES = 100000
N_EDGES = 3200000
D_EDGE = 16
D_NODE = 128
D_OUT = 128

NC = 2
NS = 16
NW = NC * NS

EPW = N_EDGES // NW
CHUNK = 800
IW = 100
KROW = CHUNK // IW
NCHUNK = EPW // CHUNK
IRPW = EPW // IW
N_PAD = 100096
STRIPE = N_PAD // NS


def _sc_scatter(x_edge, src_rows, dst_rows):
    mesh = plsc.VectorSubcoreMesh(core_axis_name="c", subcore_axis_name="s")

    @functools.partial(
        pl.kernel,
        out_type=(
            jax.ShapeDtypeStruct((N_PAD, D_EDGE), jnp.float32),
            jax.ShapeDtypeStruct((N_PAD, D_EDGE), jnp.float32),
        ),
        mesh=mesh,
        compiler_params=pltpu.CompilerParams(use_tc_tiling_on_sc=False),
        scratch_types=[
            pltpu.VMEM_SHARED((N_PAD, D_EDGE), jnp.float32),
            pltpu.VMEM((CHUNK, D_EDGE), jnp.float32),
            pltpu.VMEM((KROW, IW), jnp.int32),
            pltpu.VMEM((KROW, IW), jnp.int32),
        ],
    )
    def k(xe_hbm, si_hbm, di_hbm, out0_hbm, out1_hbm, agg, xe, si, di):
        c = lax.axis_index("c")
        s = lax.axis_index("s")
        w = s * NC + c

        def zb(i, carry):
            xe[i, :] = jnp.zeros((D_EDGE,), jnp.float32)
            return carry

        lax.fori_loop(0, CHUNK, zb, 0)
        for t in range(STRIPE // CHUNK):
            pltpu.sync_copy(xe, agg.at[pl.ds(s * STRIPE + t * CHUNK, CHUNK)])
        rem = STRIPE - (STRIPE // CHUNK) * CHUNK
        if rem:
            pltpu.sync_copy(
                xe.at[pl.ds(0, rem)],
                agg.at[pl.ds(s * STRIPE + (STRIPE // CHUNK) * CHUNK, rem)],
            )
        plsc.subcore_barrier()

        ebase = w * EPW
        ibase = w * IRPW

        def body(t, carry):
            pltpu.sync_copy(xe_hbm.at[pl.ds(ebase + t * CHUNK, CHUNK)], xe)
            pltpu.sync_copy(si_hbm.at[pl.ds(ibase + t * KROW, KROW)], si)
            pltpu.sync_copy(di_hbm.at[pl.ds(ibase + t * KROW, KROW)], di)

            def row(j, carry2):
                xsl = xe.at[pl.ds(j * IW, IW)]
                pltpu.sync_copy(xsl, agg.at[si.at[j]], add=True)
                pltpu.sync_copy(xsl, agg.at[di.at[j]], add=True)
                return carry2

            return lax.fori_loop(0, KROW, row, carry)

        lax.fori_loop(0, NCHUNK, body, 0)
        plsc.subcore_barrier()

        stripe_src = agg.at[pl.ds(s * STRIPE, STRIPE)]

        @pl.when(c == 0)
        def _():
            pltpu.sync_copy(stripe_src, out0_hbm.at[pl.ds(s * STRIPE, STRIPE)])

        @pl.when(c == 1)
        def _():
            pltpu.sync_copy(stripe_src, out1_hbm.at[pl.ds(s * STRIPE, STRIPE)])

    return k(x_edge, src_rows, dst_rows)


RB = 1000


def _tc_mlp(x_node, p0, p1, W, b2):
    def mm(x_ref, p0_ref, p1_ref, w_ref, b_ref, o_ref):
        w = w_ref[...]
        acc = jnp.dot(x_ref[...], w[:D_NODE, :], preferred_element_type=jnp.float32)
        acc += jnp.dot(p0_ref[...], w[D_NODE:, :], preferred_element_type=jnp.float32)
        acc += jnp.dot(p1_ref[...], w[D_NODE:, :], preferred_element_type=jnp.float32)
        o_ref[...] = acc + b_ref[...]

    nb = N_NODES // RB
    return pl.pallas_call(
        mm,
        grid=(nb,),
        in_specs=[
            pl.BlockSpec((RB, D_NODE), lambda i: (i, 0)),
            pl.BlockSpec((RB, D_EDGE), lambda i: (i, 0)),
            pl.BlockSpec((RB, D_EDGE), lambda i: (i, 0)),
            pl.BlockSpec((D_NODE + D_EDGE, D_OUT), lambda i: (0, 0)),
            pl.BlockSpec((1, D_OUT), lambda i: (0, 0)),
        ],
        out_specs=pl.BlockSpec((RB, D_OUT), lambda i: (i, 0)),
        out_shape=jax.ShapeDtypeStruct((N_NODES, D_OUT), jnp.float32),
    )(x_node, p0, p1, W, b2)


def kernel(x_node, x_edge, edge_index, W, b):
    src = edge_index[:, 0].reshape(N_EDGES // IW, IW)
    dst = edge_index[:, 1].reshape(N_EDGES // IW, IW)
    p0, p1 = _sc_scatter(x_edge, src, dst)
    out = _tc_mlp(x_node, p0, p1, W, b.reshape(1, D_OUT))
    return (out, x_edge, edge_index)


# speedup vs baseline: 8.3817x; 1.1524x over previous
import functools

import jax
import jax.numpy as jnp
from jax import lax
from jax.experimental import pallas as pl
from jax.experimental.pallas import tpu as pltpu
from jax.experimental.pallas import tpu_sc as plsc

N_NODES = 100000
N_EDGES = 3200000
D_EDGE = 16
D_NODE = 128
D_OUT = 128

NC = 2
NS = 16
NW = NC * NS

EPW = N_EDGES // NW
CHUNK = 400
IW = 100
KROW = CHUNK // IW
NCHUNK = EPW // CHUNK
NPAIR = NCHUNK // 2
IRPW = EPW // IW
N_PAD = 100096
STRIPE = N_PAD // NS


def _sc_scatter(x_edge, src_rows, dst_rows):
    mesh = plsc.VectorSubcoreMesh(core_axis_name="c", subcore_axis_name="s")

    @functools.partial(
        pl.kernel,
        out_type=(
            jax.ShapeDtypeStruct((N_PAD, D_EDGE), jnp.float32),
            jax.ShapeDtypeStruct((N_PAD, D_EDGE), jnp.float32),
        ),
        mesh=mesh,
        compiler_params=pltpu.CompilerParams(use_tc_tiling_on_sc=False),
        scratch_types=[
            pltpu.VMEM_SHARED((N_PAD, D_EDGE), jnp.float32),
            pltpu.VMEM((CHUNK, D_EDGE), jnp.float32),
            pltpu.VMEM((CHUNK, D_EDGE), jnp.float32),
            pltpu.VMEM((KROW, IW), jnp.int32),
            pltpu.VMEM((KROW, IW), jnp.int32),
            pltpu.VMEM((KROW, IW), jnp.int32),
            pltpu.VMEM((KROW, IW), jnp.int32),
            pltpu.SemaphoreType.DMA,
            pltpu.SemaphoreType.DMA,
            pltpu.SemaphoreType.DMA,
            pltpu.SemaphoreType.DMA,
        ],
    )
    def k(xe_hbm, si_hbm, di_hbm, out0_hbm, out1_hbm, agg,
          xe0, xe1, si0, si1, di0, di1, sl0, sl1, ss0, ss1):
        c = lax.axis_index("c")
        s = lax.axis_index("s")
        w = s * NC + c

        xe = (xe0, xe1)
        si = (si0, si1)
        di = (di0, di1)
        sl = (sl0, sl1)
        ss = (ss0, ss1)

        def zb(i, carry):
            xe0[i, :] = jnp.zeros((D_EDGE,), jnp.float32)
            return carry

        lax.fori_loop(0, CHUNK, zb, 0)
        for t in range(STRIPE // CHUNK):
            pltpu.sync_copy(xe0, agg.at[pl.ds(s * STRIPE + t * CHUNK, CHUNK)])
        rem = STRIPE - (STRIPE // CHUNK) * CHUNK
        if rem:
            pltpu.sync_copy(
                xe0.at[pl.ds(0, rem)],
                agg.at[pl.ds(s * STRIPE + (STRIPE // CHUNK) * CHUNK, rem)],
            )
        plsc.subcore_barrier()

        ebase = w * EPW
        ibase = w * IRPW

        def issue_loads(t, b):
            pltpu.async_copy(xe_hbm.at[pl.ds(ebase + t * CHUNK, CHUNK)], xe[b], sl[b])
            pltpu.async_copy(si_hbm.at[pl.ds(ibase + t * KROW, KROW)], si[b], sl[b])
            pltpu.async_copy(di_hbm.at[pl.ds(ibase + t * KROW, KROW)], di[b], sl[b])

        def drain_loads(b):
            pltpu.make_async_copy(
                xe_hbm.at[pl.ds(0, CHUNK)], xe[b], sl[b]).wait()
            pltpu.make_async_copy(
                si_hbm.at[pl.ds(0, KROW)], si[b], sl[b]).wait()
            pltpu.make_async_copy(
                di_hbm.at[pl.ds(0, KROW)], di[b], sl[b]).wait()

        def issue_scat(b):
            def row(j, carry):
                xsl = xe[b].at[pl.ds(j * IW, IW)]
                pltpu.async_copy(xsl, agg.at[si[b].at[j]], ss[b], add=True)
                pltpu.async_copy(xsl, agg.at[di[b].at[j]], ss[b], add=True)
                return carry

            lax.fori_loop(0, KROW, row, 0)

        def drain_scat(b):
            def row(j, carry):
                xsl = xe[b].at[pl.ds(0, IW)]
                pltpu.make_async_copy(xsl, agg.at[si[b].at[0]], ss[b]).wait()
                pltpu.make_async_copy(xsl, agg.at[di[b].at[0]], ss[b]).wait()
                return carry

            lax.fori_loop(0, KROW, row, 0)

        issue_loads(0, 0)

        def pair(i, carry):
            t0 = 2 * i
            drain_loads(0)
            issue_scat(0)

            @pl.when(i > 0)
            def _():
                drain_scat(1)

            issue_loads(t0 + 1, 1)
            drain_loads(1)
            issue_scat(1)
            drain_scat(0)

            @pl.when(i < NPAIR - 1)
            def _():
                issue_loads(t0 + 2, 0)

            return carry

        lax.fori_loop(0, NPAIR, pair, 0)
        drain_scat(1)
        plsc.subcore_barrier()

        stripe_src = agg.at[pl.ds(s * STRIPE, STRIPE)]

        @pl.when(c == 0)
        def _():
            pltpu.sync_copy(stripe_src, out0_hbm.at[pl.ds(s * STRIPE, STRIPE)])

        @pl.when(c == 1)
        def _():
            pltpu.sync_copy(stripe_src, out1_hbm.at[pl.ds(s * STRIPE, STRIPE)])

    return k(x_edge, src_rows, dst_rows)


RB = 1000


def _tc_mlp(x_node, p0, p1, W, b2):
    def mm(x_ref, p0_ref, p1_ref, w_ref, b_ref, o_ref):
        w = w_ref[...]
        acc = jnp.dot(x_ref[...], w[:D_NODE, :], preferred_element_type=jnp.float32)
        acc += jnp.dot(p0_ref[...], w[D_NODE:, :], preferred_element_type=jnp.float32)
        acc += jnp.dot(p1_ref[...], w[D_NODE:, :], preferred_element_type=jnp.float32)
        o_ref[...] = acc + b_ref[...]

    nb = N_NODES // RB
    return pl.pallas_call(
        mm,
        grid=(nb,),
        in_specs=[
            pl.BlockSpec((RB, D_NODE), lambda i: (i, 0)),
            pl.BlockSpec((RB, D_EDGE), lambda i: (i, 0)),
            pl.BlockSpec((RB, D_EDGE), lambda i: (i, 0)),
            pl.BlockSpec((D_NODE + D_EDGE, D_OUT), lambda i: (0, 0)),
            pl.BlockSpec((1, D_OUT), lambda i: (0, 0)),
        ],
        out_specs=pl.BlockSpec((RB, D_OUT), lambda i: (i, 0)),
        out_shape=jax.ShapeDtypeStruct((N_NODES, D_OUT), jnp.float32),
    )(x_node, p0, p1, W, b2)


def kernel(x_node, x_edge, edge_index, W, b):
    src = edge_index[:, 0].reshape(N_EDGES // IW, IW)
    dst = edge_index[:, 1].reshape(N_EDGES // IW, IW)
    p0, p1 = _sc_scatter(x_edge, src, dst)
    out = _tc_mlp(x_node, p0, p1, W, b.reshape(1, D_OUT))
    return (out, x_edge, edge_index)


# speedup vs baseline: 13.1371x; 1.5674x over previous
import functools

import jax
import jax.numpy as jnp
from jax import lax
from jax.experimental import pallas as pl
from jax.experimental.pallas import tpu as pltpu
from jax.experimental.pallas import tpu_sc as plsc

N_NODES = 100000
N_EDGES = 3200000
D_EDGE = 16
D_NODE = 128
D_OUT = 128

NC = 2
NS = 16
LANES = 16

EPC = N_EDGES // NC
CHUNK = 2000
NT = EPC // CHUNK
NTPAIR = NT // 2
GROUPS = CHUNK // LANES
UNROLL = 5
N_COLS = 100352


def _sc_scatter_t(xe_t, ei_t):
    mesh = plsc.VectorSubcoreMesh(core_axis_name="c", subcore_axis_name="s")

    @functools.partial(
        pl.kernel,
        out_type=jax.ShapeDtypeStruct((NC * NS, N_COLS), jnp.float32),
        mesh=mesh,
        compiler_params=pltpu.CompilerParams(
            use_tc_tiling_on_sc=False, needs_layout_passes=False),
        scratch_types=[
            pltpu.VMEM((N_COLS,), jnp.float32),
            pltpu.VMEM((CHUNK,), jnp.float32),
            pltpu.VMEM((CHUNK,), jnp.float32),
            pltpu.VMEM((CHUNK,), jnp.int32),
            pltpu.VMEM((CHUNK,), jnp.int32),
            pltpu.VMEM((CHUNK,), jnp.int32),
            pltpu.VMEM((CHUNK,), jnp.int32),
            pltpu.SemaphoreType.DMA,
            pltpu.SemaphoreType.DMA,
        ],
    )
    def k(xe_hbm, ei_hbm, out_hbm, acc,
          xb0, xb1, sb0, sb1, db0, db1, sl0, sl1):
        c = lax.axis_index("c")
        s = lax.axis_index("s")

        xb = (xb0, xb1)
        sb = (sb0, sb1)
        db = (db0, db1)
        sl = (sl0, sl1)

        zero = jnp.zeros((LANES,), jnp.float32)

        def zb(i, carry):
            acc[pl.ds(i * (4 * LANES), LANES)] = zero
            acc[pl.ds(i * (4 * LANES) + LANES, LANES)] = zero
            acc[pl.ds(i * (4 * LANES) + 2 * LANES, LANES)] = zero
            acc[pl.ds(i * (4 * LANES) + 3 * LANES, LANES)] = zero
            return carry

        lax.fori_loop(0, N_COLS // (4 * LANES), zb, 0)

        ebase = c * EPC

        def issue_loads(t, b):
            base = ebase + t * CHUNK
            pltpu.async_copy(xe_hbm.at[s, pl.ds(base, CHUNK)], xb[b], sl[b])
            pltpu.async_copy(ei_hbm.at[0, pl.ds(base, CHUNK)], sb[b], sl[b])
            pltpu.async_copy(ei_hbm.at[1, pl.ds(base, CHUNK)], db[b], sl[b])

        def drain_loads(b):
            pltpu.make_async_copy(xe_hbm.at[0, pl.ds(0, CHUNK)], xb[b], sl[b]).wait()
            pltpu.make_async_copy(ei_hbm.at[0, pl.ds(0, CHUNK)], sb[b], sl[b]).wait()
            pltpu.make_async_copy(ei_hbm.at[1, pl.ds(0, CHUNK)], db[b], sl[b]).wait()

        def compute(b):
            def grp(g, carry):
                for u in range(UNROLL):
                    off = g * (UNROLL * LANES) + u * LANES
                    v = xb[b][pl.ds(off, LANES)]
                    i0 = sb[b][pl.ds(off, LANES)]
                    i1 = db[b][pl.ds(off, LANES)]
                    plsc.addupdate_scatter(acc, [i0], v)
                    plsc.addupdate_scatter(acc, [i1], v)
                return carry

            lax.fori_loop(0, GROUPS // UNROLL, grp, 0)

        issue_loads(0, 0)

        def pair(i, carry):
            t0 = 2 * i
            drain_loads(0)
            issue_loads(t0 + 1, 1)
            compute(0)
            drain_loads(1)

            @pl.when(i < NTPAIR - 1)
            def _():
                issue_loads(t0 + 2, 0)

            compute(1)
            return carry

        lax.fori_loop(0, NTPAIR, pair, 0)

        pltpu.sync_copy(acc, out_hbm.at[c * NS + s])

    return k(xe_t, ei_t)


RB = 1024


def _tc_mlp(x_node, agg_t, W, b2):
    def mm(x_ref, p_ref, w_ref, b_ref, o_ref):
        w = w_ref[...]
        p = p_ref[...]
        a_t = p[:NS, :] + p[NS:, :]
        acc = jnp.dot(x_ref[...], w[:D_NODE, :], preferred_element_type=jnp.float32)
        acc += lax.dot_general(
            a_t, w[D_NODE:, :],
            dimension_numbers=(((0,), (0,)), ((), ())),
            preferred_element_type=jnp.float32,
        )
        o_ref[...] = acc + b_ref[...]

    nb = N_COLS // RB
    return pl.pallas_call(
        mm,
        grid=(nb,),
        in_specs=[
            pl.BlockSpec((RB, D_NODE), lambda i: (i, 0)),
            pl.BlockSpec((NC * NS, RB), lambda i: (0, i)),
            pl.BlockSpec((D_NODE + D_EDGE, D_OUT), lambda i: (0, 0)),
            pl.BlockSpec((1, D_OUT), lambda i: (0, 0)),
        ],
        out_specs=pl.BlockSpec((RB, D_OUT), lambda i: (i, 0)),
        out_shape=jax.ShapeDtypeStruct((N_NODES, D_OUT), jnp.float32),
    )(x_node, agg_t, W, b2)


def kernel(x_node, x_edge, edge_index, W, b):
    xe_t = x_edge.T
    ei_t = edge_index.T
    agg_t = _sc_scatter_t(xe_t, ei_t)
    out = _tc_mlp(x_node, agg_t, W, b.reshape(1, D_OUT))
    return (out, x_edge, edge_index)


# speedup vs baseline: 13.7139x; 1.0439x over previous
import functools

import jax
import jax.numpy as jnp
from jax import lax
from jax.experimental import pallas as pl
from jax.experimental.pallas import tpu as pltpu
from jax.experimental.pallas import tpu_sc as plsc

N_NODES = 100000
N_EDGES = 3200000
D_EDGE = 16
D_NODE = 128
D_OUT = 128

NC = 2
NS = 16
LANES = 16

EPC = N_EDGES // NC
CHUNK = 2000
NT = EPC // CHUNK
NTPAIR = NT // 2
GROUPS = CHUNK // LANES
UNROLL = 5
N_COLS = 100352


def _sc_scatter_t(xe_t, ei_t):
    mesh = plsc.VectorSubcoreMesh(core_axis_name="c", subcore_axis_name="s")

    @functools.partial(
        pl.kernel,
        out_type=jax.ShapeDtypeStruct((NC * NS, N_COLS), jnp.float32),
        mesh=mesh,
        compiler_params=pltpu.CompilerParams(
            use_tc_tiling_on_sc=False, needs_layout_passes=False),
        scratch_types=[
            pltpu.VMEM((N_COLS,), jnp.float32),
            pltpu.VMEM((CHUNK,), jnp.float32),
            pltpu.VMEM((CHUNK,), jnp.float32),
            pltpu.VMEM((CHUNK,), jnp.int32),
            pltpu.VMEM((CHUNK,), jnp.int32),
            pltpu.VMEM((CHUNK,), jnp.int32),
            pltpu.VMEM((CHUNK,), jnp.int32),
            pltpu.SemaphoreType.DMA,
            pltpu.SemaphoreType.DMA,
        ],
    )
    def k(xe_hbm, ei_hbm, out_hbm, acc,
          xb0, xb1, sb0, sb1, db0, db1, sl0, sl1):
        c = lax.axis_index("c")
        s = lax.axis_index("s")

        xb = (xb0, xb1)
        sb = (sb0, sb1)
        db = (db0, db1)
        sl = (sl0, sl1)

        zero = jnp.zeros((LANES,), jnp.float32)

        def zb(i, carry):
            acc[pl.ds(i * (4 * LANES), LANES)] = zero
            acc[pl.ds(i * (4 * LANES) + LANES, LANES)] = zero
            acc[pl.ds(i * (4 * LANES) + 2 * LANES, LANES)] = zero
            acc[pl.ds(i * (4 * LANES) + 3 * LANES, LANES)] = zero
            return carry

        lax.fori_loop(0, N_COLS // (4 * LANES), zb, 0)

        ebase = c * EPC

        def issue_loads(t, b):
            base = ebase + t * CHUNK
            pltpu.async_copy(xe_hbm.at[s, pl.ds(base, CHUNK)], xb[b], sl[b])
            pltpu.async_copy(ei_hbm.at[0, pl.ds(base, CHUNK)], sb[b], sl[b])
            pltpu.async_copy(ei_hbm.at[1, pl.ds(base, CHUNK)], db[b], sl[b])

        def drain_loads(b):
            pltpu.make_async_copy(xe_hbm.at[0, pl.ds(0, CHUNK)], xb[b], sl[b]).wait()
            pltpu.make_async_copy(ei_hbm.at[0, pl.ds(0, CHUNK)], sb[b], sl[b]).wait()
            pltpu.make_async_copy(ei_hbm.at[1, pl.ds(0, CHUNK)], db[b], sl[b]).wait()

        def compute(b):
            @plsc.parallel_loop(0, GROUPS, 1, unroll=UNROLL)
            def grp(g):
                off = g * LANES
                v = xb[b][pl.ds(off, LANES)]
                i0 = sb[b][pl.ds(off, LANES)]
                i1 = db[b][pl.ds(off, LANES)]
                plsc.addupdate_scatter(acc, [i0], v)
                plsc.addupdate_scatter(acc, [i1], v)

        issue_loads(0, 0)

        def pair(i, carry):
            t0 = 2 * i
            drain_loads(0)
            issue_loads(t0 + 1, 1)
            compute(0)
            drain_loads(1)

            @pl.when(i < NTPAIR - 1)
            def _():
                issue_loads(t0 + 2, 0)

            compute(1)
            return carry

        lax.fori_loop(0, NTPAIR, pair, 0)

        pltpu.sync_copy(acc, out_hbm.at[c * NS + s])

    return k(xe_t, ei_t)


RB = 1024


def _tc_mlp(x_node, agg_t, W, b2):
    def mm(x_ref, p_ref, w_ref, b_ref, o_ref):
        w = w_ref[...]
        p = p_ref[...]
        a_t = p[:NS, :] + p[NS:, :]
        acc = jnp.dot(x_ref[...], w[:D_NODE, :], preferred_element_type=jnp.float32)
        acc += lax.dot_general(
            a_t, w[D_NODE:, :],
            dimension_numbers=(((0,), (0,)), ((), ())),
            preferred_element_type=jnp.float32,
        )
        o_ref[...] = acc + b_ref[...]

    nb = N_COLS // RB
    return pl.pallas_call(
        mm,
        grid=(nb,),
        in_specs=[
            pl.BlockSpec((RB, D_NODE), lambda i: (i, 0)),
            pl.BlockSpec((NC * NS, RB), lambda i: (0, i)),
            pl.BlockSpec((D_NODE + D_EDGE, D_OUT), lambda i: (0, 0)),
            pl.BlockSpec((1, D_OUT), lambda i: (0, 0)),
        ],
        out_specs=pl.BlockSpec((RB, D_OUT), lambda i: (i, 0)),
        out_shape=jax.ShapeDtypeStruct((N_NODES, D_OUT), jnp.float32),
    )(x_node, agg_t, W, b2)


def kernel(x_node, x_edge, edge_index, W, b):
    xe_t = x_edge.T
    ei_t = edge_index.T
    agg_t = _sc_scatter_t(xe_t, ei_t)
    out = _tc_mlp(x_node, agg_t, W, b.reshape(1, D_OUT))
    return (out, x_edge, edge_index)


# speedup vs baseline: 19.6890x; 1.4357x over previous
import functools

import jax
import jax.numpy as jnp
from jax import lax
from jax.experimental import pallas as pl
from jax.experimental.pallas import tpu as pltpu
from jax.experimental.pallas import tpu_sc as plsc

N_NODES = 100000
N_EDGES = 3200000
D_EDGE = 16
D_NODE = 128
D_OUT = 128

NC = 2
NS = 16
LANES = 16
FB = 8
EB = N_EDGES // 128
EBC = EB // NC

CB = 25
CHUNK = CB * 128
NT = EBC // CB
NTPAIR = NT // 2
UNROLL = 5
N_COLS = 100352


def _sc_scatter_t(xe4, ei4):
    mesh = plsc.VectorSubcoreMesh(core_axis_name="c", subcore_axis_name="s")

    @functools.partial(
        pl.kernel,
        out_type=jax.ShapeDtypeStruct((NC * NS, N_COLS), jnp.float32),
        mesh=mesh,
        compiler_params=pltpu.CompilerParams(
            use_tc_tiling_on_sc=False, needs_layout_passes=False),
        scratch_types=[
            pltpu.VMEM((N_COLS,), jnp.float32),
            pltpu.VMEM((CB, 128), jnp.float32),
            pltpu.VMEM((CB, 128), jnp.float32),
            pltpu.VMEM((CB, 128), jnp.int32),
            pltpu.VMEM((CB, 128), jnp.int32),
            pltpu.VMEM((CB, 128), jnp.int32),
            pltpu.VMEM((CB, 128), jnp.int32),
            pltpu.SemaphoreType.DMA,
            pltpu.SemaphoreType.DMA,
        ],
    )
    def k(xe_hbm, ei_hbm, out_hbm, acc,
          xb0, xb1, sb0, sb1, db0, db1, sl0, sl1):
        c = lax.axis_index("c")
        s = lax.axis_index("s")
        fb = s // FB
        fr = s % FB

        xb = (xb0, xb1)
        sb = (sb0, sb1)
        db = (db0, db1)
        sl = (sl0, sl1)

        zero = jnp.zeros((LANES,), jnp.float32)

        def zb(i, carry):
            acc[pl.ds(i * (4 * LANES), LANES)] = zero
            acc[pl.ds(i * (4 * LANES) + LANES, LANES)] = zero
            acc[pl.ds(i * (4 * LANES) + 2 * LANES, LANES)] = zero
            acc[pl.ds(i * (4 * LANES) + 3 * LANES, LANES)] = zero
            return carry

        lax.fori_loop(0, N_COLS // (4 * LANES), zb, 0)

        jcore = c * EBC

        def issue_loads(t, b):
            jbase = jcore + t * CB
            pltpu.async_copy(
                xe_hbm.at[fb, pl.ds(jbase, CB), fr, :], xb[b], sl[b])
            pltpu.async_copy(ei_hbm.at[pl.ds(jbase, CB), 0, :], sb[b], sl[b])
            pltpu.async_copy(ei_hbm.at[pl.ds(jbase, CB), 1, :], db[b], sl[b])

        def drain_loads(b):
            pltpu.make_async_copy(
                xe_hbm.at[0, pl.ds(0, CB), 0, :], xb[b], sl[b]).wait()
            pltpu.make_async_copy(
                ei_hbm.at[pl.ds(0, CB), 0, :], sb[b], sl[b]).wait()
            pltpu.make_async_copy(
                ei_hbm.at[pl.ds(0, CB), 1, :], db[b], sl[b]).wait()

        def compute(b):
            @plsc.parallel_loop(0, CB, 1, unroll=UNROLL)
            def grp(jj):
                for u in range(FB):
                    sl16 = pl.ds(u * LANES, LANES)
                    v = xb[b][jj, sl16]
                    i0 = sb[b][jj, sl16]
                    i1 = db[b][jj, sl16]
                    plsc.addupdate_scatter(acc, [i0], v)
                    plsc.addupdate_scatter(acc, [i1], v)

        issue_loads(0, 0)

        def pair(i, carry):
            t0 = 2 * i
            drain_loads(0)
            issue_loads(t0 + 1, 1)
            compute(0)
            drain_loads(1)

            @pl.when(i < NTPAIR - 1)
            def _():
                issue_loads(t0 + 2, 0)

            compute(1)
            return carry

        lax.fori_loop(0, NTPAIR, pair, 0)

        pltpu.sync_copy(acc, out_hbm.at[c * NS + s])

    return k(xe4, ei4)


RB = 1024


def _tc_mlp(x_node, agg_t, W, b2):
    def mm(x_ref, p_ref, w_ref, b_ref, o_ref):
        w = w_ref[...]
        p = p_ref[...]
        a_t = p[:NS, :] + p[NS:, :]
        acc = jnp.dot(x_ref[...], w[:D_NODE, :], preferred_element_type=jnp.float32)
        acc += lax.dot_general(
            a_t, w[D_NODE:, :],
            dimension_numbers=(((0,), (0,)), ((), ())),
            preferred_element_type=jnp.float32,
        )
        o_ref[...] = acc + b_ref[...]

    nb = N_COLS // RB
    return pl.pallas_call(
        mm,
        grid=(nb,),
        in_specs=[
            pl.BlockSpec((RB, D_NODE), lambda i: (i, 0)),
            pl.BlockSpec((NC * NS, RB), lambda i: (0, i)),
            pl.BlockSpec((D_NODE + D_EDGE, D_OUT), lambda i: (0, 0)),
            pl.BlockSpec((1, D_OUT), lambda i: (0, 0)),
        ],
        out_specs=pl.BlockSpec((RB, D_OUT), lambda i: (i, 0)),
        out_shape=jax.ShapeDtypeStruct((N_NODES, D_OUT), jnp.float32),
    )(x_node, agg_t, W, b2)


def kernel(x_node, x_edge, edge_index, W, b):
    xe4 = x_edge.T.reshape(NC, FB, EB, 128).transpose(0, 2, 1, 3)
    ei4 = edge_index.T.reshape(2, EB, 128).transpose(1, 0, 2)
    agg_t = _sc_scatter_t(xe4, ei4)
    out = _tc_mlp(x_node, agg_t, W, b.reshape(1, D_OUT))
    return (out, x_edge, edge_index)


# speedup vs baseline: 21.3901x; 1.0864x over previous
import functools

import jax
import jax.numpy as jnp
from jax import lax
from jax.experimental import pallas as pl
from jax.experimental.pallas import tpu as pltpu
from jax.experimental.pallas import tpu_sc as plsc

N_NODES = 100000
N_EDGES = 3200000
D_EDGE = 16
D_NODE = 128
D_OUT = 128

NC = 2
NS = 16
LANES = 16
FB = 8
EB = N_EDGES // 128
EBC = EB // NC

CB = 25
CHUNK = CB * 128
NT = EBC // CB
NTPAIR = NT // 2
UNROLL = 5
N_COLS = 100352


def _sc_scatter_t(xe4, ei4):
    mesh = plsc.VectorSubcoreMesh(core_axis_name="c", subcore_axis_name="s")

    @functools.partial(
        pl.kernel,
        out_type=(
            jax.ShapeDtypeStruct((NC * NS, N_COLS), jnp.float32),
            jax.ShapeDtypeStruct((NC, EB, FB, 128), jnp.float32),
            jax.ShapeDtypeStruct((EB, 2, 128), jnp.int32),
        ),
        mesh=mesh,
        compiler_params=pltpu.CompilerParams(
            use_tc_tiling_on_sc=False, needs_layout_passes=False),
        scratch_types=[
            pltpu.VMEM((N_COLS,), jnp.float32),
            pltpu.VMEM((CB, 128), jnp.float32),
            pltpu.VMEM((CB, 128), jnp.float32),
            pltpu.VMEM((CB, 128), jnp.int32),
            pltpu.VMEM((CB, 128), jnp.int32),
            pltpu.VMEM((CB, 128), jnp.int32),
            pltpu.VMEM((CB, 128), jnp.int32),
            pltpu.SemaphoreType.DMA,
            pltpu.SemaphoreType.DMA,
            pltpu.SemaphoreType.DMA,
            pltpu.SemaphoreType.DMA,
        ],
    )
    def k(xe_hbm, ei_hbm, out_hbm, oxe_hbm, oei_hbm, acc,
          xb0, xb1, sb0, sb1, db0, db1, sl0, sl1, se0, se1):
        c = lax.axis_index("c")
        s = lax.axis_index("s")
        fb = s // FB
        fr = s % FB

        xb = (xb0, xb1)
        sb = (sb0, sb1)
        db = (db0, db1)
        sl = (sl0, sl1)
        se = (se0, se1)

        zero = jnp.zeros((LANES,), jnp.float32)

        def zb(i, carry):
            acc[pl.ds(i * (4 * LANES), LANES)] = zero
            acc[pl.ds(i * (4 * LANES) + LANES, LANES)] = zero
            acc[pl.ds(i * (4 * LANES) + 2 * LANES, LANES)] = zero
            acc[pl.ds(i * (4 * LANES) + 3 * LANES, LANES)] = zero
            return carry

        lax.fori_loop(0, N_COLS // (4 * LANES), zb, 0)

        jcore = c * EBC

        def issue_loads(t, b):
            jbase = jcore + t * CB
            pltpu.async_copy(
                xe_hbm.at[fb, pl.ds(jbase, CB), fr, :], xb[b], sl[b])
            pltpu.async_copy(ei_hbm.at[pl.ds(jbase, CB), 0, :], sb[b], sl[b])
            pltpu.async_copy(ei_hbm.at[pl.ds(jbase, CB), 1, :], db[b], sl[b])

        def drain_loads(b):
            pltpu.make_async_copy(
                xe_hbm.at[0, pl.ds(0, CB), 0, :], xb[b], sl[b]).wait()
            pltpu.make_async_copy(
                ei_hbm.at[pl.ds(0, CB), 0, :], sb[b], sl[b]).wait()
            pltpu.make_async_copy(
                ei_hbm.at[pl.ds(0, CB), 1, :], db[b], sl[b]).wait()

        def issue_echo(t, b):
            jbase = jcore + t * CB
            pltpu.async_copy(
                xb[b], oxe_hbm.at[fb, pl.ds(jbase, CB), fr, :], se[b])

            @pl.when(s == 0)
            def _():
                pltpu.async_copy(
                    sb[b], oei_hbm.at[pl.ds(jbase, CB), 0, :], se[b])
                pltpu.async_copy(
                    db[b], oei_hbm.at[pl.ds(jbase, CB), 1, :], se[b])

        def drain_echo(b):
            pltpu.make_async_copy(
                xb[b], oxe_hbm.at[0, pl.ds(0, CB), 0, :], se[b]).wait()

            @pl.when(s == 0)
            def _():
                pltpu.make_async_copy(
                    sb[b], oei_hbm.at[pl.ds(0, CB), 0, :], se[b]).wait()
                pltpu.make_async_copy(
                    db[b], oei_hbm.at[pl.ds(0, CB), 1, :], se[b]).wait()

        def compute(b):
            @plsc.parallel_loop(0, CB, 1, unroll=UNROLL)
            def grp(jj):
                for u in range(FB):
                    sl16 = pl.ds(u * LANES, LANES)
                    v = xb[b][jj, sl16]
                    i0 = sb[b][jj, sl16]
                    i1 = db[b][jj, sl16]
                    plsc.addupdate_scatter(acc, [i0], v)
                    plsc.addupdate_scatter(acc, [i1], v)

        issue_loads(0, 0)

        def pair(i, carry):
            t0 = 2 * i
            drain_loads(0)
            issue_echo(t0, 0)

            @pl.when(i > 0)
            def _():
                drain_echo(1)

            issue_loads(t0 + 1, 1)
            compute(0)
            drain_loads(1)
            issue_echo(t0 + 1, 1)
            drain_echo(0)

            @pl.when(i < NTPAIR - 1)
            def _():
                issue_loads(t0 + 2, 0)

            compute(1)
            return carry

        lax.fori_loop(0, NTPAIR, pair, 0)
        drain_echo(1)

        pltpu.sync_copy(acc, out_hbm.at[c * NS + s])

    return k(xe4, ei4)


RB = 1024


def _tc_mlp(x_node, agg_t, W, b2):
    def mm(x_ref, p_ref, w_ref, b_ref, o_ref):
        w = w_ref[...]
        p = p_ref[...]
        a_t = p[:NS, :] + p[NS:, :]
        acc = jnp.dot(x_ref[...], w[:D_NODE, :], preferred_element_type=jnp.float32)
        acc += lax.dot_general(
            a_t, w[D_NODE:, :],
            dimension_numbers=(((0,), (0,)), ((), ())),
            preferred_element_type=jnp.float32,
        )
        o_ref[...] = acc + b_ref[...]

    nb = N_COLS // RB
    return pl.pallas_call(
        mm,
        grid=(nb,),
        in_specs=[
            pl.BlockSpec((RB, D_NODE), lambda i: (i, 0)),
            pl.BlockSpec((NC * NS, RB), lambda i: (0, i)),
            pl.BlockSpec((D_NODE + D_EDGE, D_OUT), lambda i: (0, 0)),
            pl.BlockSpec((1, D_OUT), lambda i: (0, 0)),
        ],
        out_specs=pl.BlockSpec((RB, D_OUT), lambda i: (i, 0)),
        out_shape=jax.ShapeDtypeStruct((N_NODES, D_OUT), jnp.float32),
    )(x_node, agg_t, W, b2)


def kernel(x_node, x_edge, edge_index, W, b):
    xe4 = x_edge.T.reshape(NC, FB, EB, 128).transpose(0, 2, 1, 3)
    ei4 = edge_index.T.reshape(2, EB, 128).transpose(1, 0, 2)
    agg_t, xe4_out, ei4_out = _sc_scatter_t(xe4, ei4)
    out = _tc_mlp(x_node, agg_t, W, b.reshape(1, D_OUT))
    x_edge_out = xe4_out.transpose(0, 2, 1, 3).reshape(D_EDGE, N_EDGES).T
    edge_index_out = ei4_out.transpose(1, 0, 2).reshape(2, N_EDGES).T
    return (out, x_edge_out, edge_index_out)


# speedup vs baseline: 21.4112x; 1.0010x over previous
import functools

import jax
import jax.numpy as jnp
from jax import lax
from jax.experimental import pallas as pl
from jax.experimental.pallas import tpu as pltpu
from jax.experimental.pallas import tpu_sc as plsc

N_NODES = 100000
N_EDGES = 3200000
D_EDGE = 16
D_NODE = 128
D_OUT = 128

NC = 2
NS = 16
LANES = 16
FB = 8
EB = N_EDGES // 128
EBC = EB // NC

CB = 25
CHUNK = CB * 128
NT = EBC // CB
NTPAIR = NT // 2
UNROLL = 5
N_COLS = 100352


def _sc_scatter_t(xe4, ei4):
    mesh = plsc.VectorSubcoreMesh(core_axis_name="c", subcore_axis_name="s")

    @functools.partial(
        pl.kernel,
        out_type=(
            jax.ShapeDtypeStruct((NC * NS, N_COLS), jnp.float32),
            jax.ShapeDtypeStruct((NC, EB, FB, 128), jnp.float32),
            jax.ShapeDtypeStruct((EB, 2, 128), jnp.int32),
        ),
        mesh=mesh,
        compiler_params=pltpu.CompilerParams(
            use_tc_tiling_on_sc=False, needs_layout_passes=False),
        scratch_types=[
            pltpu.VMEM((N_COLS,), jnp.float32),
            pltpu.VMEM((CB, 128), jnp.float32),
            pltpu.VMEM((CB, 128), jnp.float32),
            pltpu.VMEM((CB, 2, 128), jnp.int32),
            pltpu.VMEM((CB, 2, 128), jnp.int32),
            pltpu.SemaphoreType.DMA,
            pltpu.SemaphoreType.DMA,
            pltpu.SemaphoreType.DMA,
            pltpu.SemaphoreType.DMA,
        ],
    )
    def k(xe_hbm, ei_hbm, out_hbm, oxe_hbm, oei_hbm, acc,
          xb0, xb1, ib0, ib1, sl0, sl1, se0, se1):
        c = lax.axis_index("c")
        s = lax.axis_index("s")
        fb = s // FB
        fr = s % FB

        xb = (xb0, xb1)
        ib = (ib0, ib1)
        sl = (sl0, sl1)
        se = (se0, se1)

        zero = jnp.zeros((LANES,), jnp.float32)

        def zb(i, carry):
            acc[pl.ds(i * (4 * LANES), LANES)] = zero
            acc[pl.ds(i * (4 * LANES) + LANES, LANES)] = zero
            acc[pl.ds(i * (4 * LANES) + 2 * LANES, LANES)] = zero
            acc[pl.ds(i * (4 * LANES) + 3 * LANES, LANES)] = zero
            return carry

        lax.fori_loop(0, N_COLS // (4 * LANES), zb, 0)

        jcore = c * EBC

        def issue_loads(t, b):
            jbase = jcore + t * CB
            pltpu.async_copy(
                xe_hbm.at[fb, pl.ds(jbase, CB), fr, :], xb[b], sl[b])
            pltpu.async_copy(ei_hbm.at[pl.ds(jbase, CB)], ib[b], sl[b])

        def drain_loads(b):
            pltpu.make_async_copy(
                xe_hbm.at[0, pl.ds(0, CB), 0, :], xb[b], sl[b]).wait()
            pltpu.make_async_copy(
                ei_hbm.at[pl.ds(0, CB)], ib[b], sl[b]).wait()

        def issue_echo(t, b):
            jbase = jcore + t * CB
            pltpu.async_copy(
                xb[b], oxe_hbm.at[fb, pl.ds(jbase, CB), fr, :], se[b])

            @pl.when(s == 0)
            def _():
                pltpu.async_copy(ib[b], oei_hbm.at[pl.ds(jbase, CB)], se[b])

        def drain_echo(b):
            pltpu.make_async_copy(
                xb[b], oxe_hbm.at[0, pl.ds(0, CB), 0, :], se[b]).wait()

            @pl.when(s == 0)
            def _():
                pltpu.make_async_copy(
                    ib[b], oei_hbm.at[pl.ds(0, CB)], se[b]).wait()

        def compute(b):
            @plsc.parallel_loop(0, CB, 1, unroll=UNROLL)
            def grp(jj):
                for u in range(FB):
                    sl16 = pl.ds(u * LANES, LANES)
                    v = xb[b][jj, sl16]
                    i0 = ib[b][jj, 0, sl16]
                    i1 = ib[b][jj, 1, sl16]
                    plsc.addupdate_scatter(acc, [i0], v)
                    plsc.addupdate_scatter(acc, [i1], v)

        issue_loads(0, 0)

        def pair(i, carry):
            t0 = 2 * i
            drain_loads(0)
            issue_echo(t0, 0)

            @pl.when(i > 0)
            def _():
                drain_echo(1)

            issue_loads(t0 + 1, 1)
            compute(0)
            drain_loads(1)
            issue_echo(t0 + 1, 1)
            drain_echo(0)

            @pl.when(i < NTPAIR - 1)
            def _():
                issue_loads(t0 + 2, 0)

            compute(1)
            return carry

        lax.fori_loop(0, NTPAIR, pair, 0)
        drain_echo(1)

        pltpu.sync_copy(acc, out_hbm.at[c * NS + s])

    return k(xe4, ei4)


RB = 1024


def _tc_mlp(x_node, agg_t, W, b2):
    def mm(x_ref, p_ref, w_ref, b_ref, o_ref):
        w = w_ref[...]
        p = p_ref[...]
        a_t = p[:NS, :] + p[NS:, :]
        acc = jnp.dot(x_ref[...], w[:D_NODE, :], preferred_element_type=jnp.float32)
        acc += lax.dot_general(
            a_t, w[D_NODE:, :],
            dimension_numbers=(((0,), (0,)), ((), ())),
            preferred_element_type=jnp.float32,
        )
        o_ref[...] = acc + b_ref[...]

    nb = N_COLS // RB
    return pl.pallas_call(
        mm,
        grid=(nb,),
        in_specs=[
            pl.BlockSpec((RB, D_NODE), lambda i: (i, 0)),
            pl.BlockSpec((NC * NS, RB), lambda i: (0, i)),
            pl.BlockSpec((D_NODE + D_EDGE, D_OUT), lambda i: (0, 0)),
            pl.BlockSpec((1, D_OUT), lambda i: (0, 0)),
        ],
        out_specs=pl.BlockSpec((RB, D_OUT), lambda i: (i, 0)),
        out_shape=jax.ShapeDtypeStruct((N_NODES, D_OUT), jnp.float32),
    )(x_node, agg_t, W, b2)


def kernel(x_node, x_edge, edge_index, W, b):
    xe4 = x_edge.T.reshape(NC, FB, EB, 128).transpose(0, 2, 1, 3)
    ei4 = edge_index.T.reshape(2, EB, 128).transpose(1, 0, 2)
    agg_t, xe4_out, ei4_out = _sc_scatter_t(xe4, ei4)
    out = _tc_mlp(x_node, agg_t, W, b.reshape(1, D_OUT))
    x_edge_out = xe4_out.transpose(0, 2, 1, 3).reshape(D_EDGE, N_EDGES).T
    edge_index_out = ei4_out.transpose(1, 0, 2).reshape(2, N_EDGES).T
    return (out, x_edge_out, edge_index_out)
